# baseline (speedup 1.0000x reference)
import jax
import jax.numpy as jnp
from jax import lax
from jax.experimental import pallas as pl
from jax.experimental.pallas import tpu as pltpu
from jax.experimental.pallas import tpu_sc as plsc

_OBS_VOCAB = 1000000
_NUM_ACTIONS = 8
_ACTION_DIM = 16
_EMBED_DIM = 64
_B = 1024
_L = 200
_SEQ = _L + _NUM_ACTIONS

_NC, _NS, _LANES = 2, 16, 16
_NW = _NC * _NS
_B_PER_W = _B // _NW
_CIDX = _B_PER_W * _SEQ
_CPAD = _CIDX + 16

_GSPLITS = ((0, 104), (104, 96), (_L, _NUM_ACTIONS))

_NV = _EMBED_DIM // _LANES


def _body(tok_hbm, act_hbm, obs2_hbm, atab2_hbm, out_hbm,
          cidx_v, pidx_v, act_v, p0_v, p1_v, p2_v, rows0_v, rows1_v,
          g0, g1, g2, s0, s1):
    pbufs = (p0_v, p1_v, p2_v)
    rbufs = (rows0_v, rows1_v)
    gsems = (g0, g1, g2)
    ssems = (s0, s1)

    cid = lax.axis_index("c")
    sid = lax.axis_index("s")
    wid = sid * _NC + cid
    b0 = wid * _B_PER_W

    pltpu.sync_copy(act_hbm.at[pl.ds(b0 * _NUM_ACTIONS, 256)],
                    act_v.at[pl.ds(0, 256)])
    offs = lax.bitwise_and(lax.iota(jnp.int32, _LANES), _NUM_ACTIONS - 1) \
        * _ACTION_DIM
    for j in range(_B_PER_W):
        av = act_v[pl.ds(j * _NUM_ACTIONS, _LANES)] + offs
        cidx_v[pl.ds(j * _SEQ + _L, _LANES)] = av
        pltpu.sync_copy(tok_hbm.at[pl.ds((b0 + j) * _L, _L)],
                        cidx_v.at[pl.ds(j * _SEQ, _L)])

    for r in range(_CIDX // _LANES):
        sl = pl.ds(r * _LANES, _LANES)
        pidx_v[sl] = lax.shift_right_logical(cidx_v[sl], 1)

    def gathers(j, buf):
        d = []
        for off, width in _GSPLITS[:2]:
            d.append(pltpu.make_async_copy(
                obs2_hbm.at[pidx_v.at[pl.ds(j * _SEQ + off, width)]],
                pbufs[buf].at[pl.ds(off, width)], gsems[buf]))
        off, width = _GSPLITS[2]
        d.append(pltpu.make_async_copy(
            atab2_hbm.at[pidx_v.at[pl.ds(j * _SEQ + off, width)]],
            pbufs[buf].at[pl.ds(off, width)], gsems[buf]))
        return d

    def store(j, buf):
        return pltpu.make_async_copy(rbufs[buf], out_hbm.at[b0 + j],
                                     ssems[buf])

    def extract(j, pu, ru):
        pb, rb = pbufs[pu], rbufs[ru]

        @pl.loop(0, _SEQ // _LANES)
        def _grp(g):
            hvec = lax.bitwise_and(cidx_v[pl.ds(j * _SEQ + g * _LANES,
                                                _LANES)], 1)
            for l in range(_LANES):
                off = hvec[l] * _EMBED_DIM
                prow = g * _LANES + l
                rrow = g * (_LANES // 2) + (l >> 1)
                cbase = (l & 1) * _EMBED_DIM
                for k in range(_NV):
                    rb[rrow, pl.ds(cbase + k * _LANES, _LANES)] = \
                        pb[prow, pl.ds(off + k * _LANES, _LANES)]

    def step(j, pu, ru, prime):
        for dsc in gathers(j, pu):
            dsc.wait()
        if prime:
            @pl.when(j + 2 < _B_PER_W)
            def _issue_next():
                for dsc in gathers(j + 2, (pu + 2) % 3):
                    dsc.start()

        @pl.when(j >= 2)
        def _drain_store():
            store(j - 2, ru).wait()

        extract(j, pu, ru)
        store(j, ru).start()

    for dsc in gathers(0, 0):
        dsc.start()
    for dsc in gathers(1, 1):
        dsc.start()

    @pl.loop(0, (_B_PER_W - 2) // 6)
    def _batch(i):
        for u in range(6):
            j = i * 6 + u
            step(j, u % 3, u % 2, True)

    step(_B_PER_W - 2, (_B_PER_W - 2) % 3, 0, False)
    step(_B_PER_W - 1, (_B_PER_W - 1) % 3, 1, False)

    store(_B_PER_W - 2, 0).wait()
    store(_B_PER_W - 1, 1).wait()


@jax.jit
def kernel(tokens, action, obs_table, action_table):
    mesh = plsc.VectorSubcoreMesh(core_axis_name="c", subcore_axis_name="s")
    run = pl.kernel(
        _body,
        out_type=jax.ShapeDtypeStruct((_B, _SEQ * _EMBED_DIM // 128, 128),
                                      jnp.float32),
        mesh=mesh,
        scratch_types=[
            pltpu.VMEM((_CPAD,), jnp.int32),
            pltpu.VMEM((_CPAD,), jnp.int32),
            pltpu.VMEM((272,), jnp.int32),
            pltpu.VMEM((_SEQ, 128), jnp.float32),
            pltpu.VMEM((_SEQ, 128), jnp.float32),
            pltpu.VMEM((_SEQ, 128), jnp.float32),
            pltpu.VMEM((_SEQ * _EMBED_DIM // 128, 128), jnp.float32),
            pltpu.VMEM((_SEQ * _EMBED_DIM // 128, 128), jnp.float32),
            *[pltpu.SemaphoreType.DMA for _ in range(5)],
        ],
        compiler_params=pltpu.CompilerParams(use_tc_tiling_on_sc=True),
    )
    out3 = run(tokens.astype(jnp.int32).reshape(-1),
               action.astype(jnp.int32).reshape(-1),
               obs_table.reshape(_OBS_VOCAB // 2, 2 * _EMBED_DIM),
               action_table.reshape(_NUM_ACTIONS * _ACTION_DIM // 2,
                                    2 * _EMBED_DIM))
    return out3.reshape(_B, _SEQ, _EMBED_DIM)
